# baseline (speedup 1.0000x reference)
import functools

import jax
import jax.numpy as jnp
from jax import lax
from jax.experimental import pallas as pl
from jax.experimental.pallas import tpu as pltpu
from jax.experimental.pallas import tpu_sc as plsc

_NC = 2
_NS = 16
_NW = _NC * _NS
_CH = 400
_D = 64


def _table_copy(w, src, dst, n_chunks, buf0, buf1, si0, si1, so0, so1):
    bufs = (buf0, buf1)
    sins = (si0, si1)
    souts = (so0, so1)
    n_iter = (n_chunks + _NW - 1) // _NW

    def in_copy(c, b):
        return pltpu.make_async_copy(src.at[pl.ds(c * _CH, _CH)], bufs[b], sins[b])

    def out_copy(c, b):
        return pltpu.make_async_copy(bufs[b], dst.at[pl.ds(c * _CH, _CH)], souts[b])

    for j in range(n_iter):
        b = j & 1
        c = w + _NW * j

        @pl.when(c < n_chunks)
        def _():
            if j >= 2:
                out_copy(c - 2 * _NW, b).wait()
            in_copy(c, b).start()
            in_copy(c, b).wait()
            out_copy(c, b).start()

    out_copy(w, 0).wait()
    out_copy(w, 1).wait()


def _copy_kernel(u_in, i_in, u_out, i_out, buf0, buf1, si0, si1, so0, so1):
    w = lax.axis_index("s") * _NC + lax.axis_index("c")
    nu = u_in.shape[0] // _CH
    ni = i_in.shape[0] // _CH
    _table_copy(w, u_in, u_out, nu, buf0, buf1, si0, si1, so0, so1)
    _table_copy(w, i_in, i_out, ni, buf0, buf1, si0, si1, so0, so1)


def kernel(edge_index, user_weight, item_weight):
    mesh = plsc.VectorSubcoreMesh(core_axis_name="c", subcore_axis_name="s")
    run = functools.partial(
        pl.kernel,
        mesh=mesh,
        out_type=[
            jax.ShapeDtypeStruct(user_weight.shape, user_weight.dtype),
            jax.ShapeDtypeStruct(item_weight.shape, item_weight.dtype),
        ],
        scratch_types=[
            pltpu.VMEM((_CH, _D), jnp.float32),
            pltpu.VMEM((_CH, _D), jnp.float32),
            pltpu.SemaphoreType.DMA,
            pltpu.SemaphoreType.DMA,
            pltpu.SemaphoreType.DMA,
            pltpu.SemaphoreType.DMA,
        ],
    )(_copy_kernel)
    u_out, i_out = run(user_weight, item_weight)
    return (u_out, i_out)
